# Initial kernel scaffold; baseline (speedup 1.0000x reference)
import jax
import jax.numpy as jnp
from jax.experimental import pallas as pl


def kernel(x, W_in, b_in, W_hid, b_hid, W_fc, b_fc):
    raise NotImplementedError("write your pallas kernel here")


# speedup vs baseline: 3.8526x; 3.8526x over previous
import numpy as np
import jax
import jax.numpy as jnp
from jax.experimental import pallas as pl

_C = 32
_LAYER_SIZES = [4, 8, 8, 4]


def _dag():
    rng = np.random.RandomState(0)
    layers = []
    nid = 0
    for s in _LAYER_SIZES:
        layers.append(list(range(nid, nid + s)))
        nid += s
    parents = {}
    for l in range(1, len(layers)):
        for n in layers[l]:
            parents[n] = sorted(
                rng.choice(layers[l - 1], size=2, replace=False).tolist()
            )
    return layers, parents


_LAYERS, _PARENTS = _dag()

_PLACEMENTS = []
_HID_RANGES = []
_hid = 0
for _l in range(1, len(_LAYERS)):
    _start_prev = _LAYERS[_l - 1][0]
    _pls = []
    _h0 = _hid
    for _j, _n in enumerate(_LAYERS[_l]):
        for _p in _PARENTS[_n]:
            _pls.append((_j, _p - _start_prev, _hid))
        _hid += 1
    _PLACEMENTS.append(_pls)
    _HID_RANGES.append((_h0, _hid))


def _body(x_ref, wi_ref, bi_ref, w1_ref, b1_ref, w2_ref, b2_ref,
          w3_ref, b3_ref, wfb_ref, bfc_ref, out_ref):
    n = x_ref.shape[-1]
    xb = x_ref[0]
    a = jnp.dot(wi_ref[...], xb, preferred_element_type=jnp.float32)
    a = jnp.maximum(a + bi_ref[...], 0.0)
    a = jnp.dot(w1_ref[...], a, preferred_element_type=jnp.float32)
    a = jnp.maximum(a + b1_ref[...], 0.0)
    a = jnp.dot(w2_ref[...], a, preferred_element_type=jnp.float32)
    a = jnp.maximum(a + b2_ref[...], 0.0)
    a = jnp.dot(w3_ref[...], a, preferred_element_type=jnp.float32)
    a = jnp.maximum(a + b3_ref[...], 0.0)
    pooled = jnp.sum(a, axis=1, keepdims=True) * (1.0 / n)
    logits = jax.lax.dot_general(
        pooled, wfb_ref[...], (((0,), (0,)), ((), ())),
        preferred_element_type=jnp.float32)
    out_ref[0] = logits + bfc_ref[...]


def kernel(x, W_in, b_in, W_hid, b_hid, W_fc, b_fc):
    B, IC, H, W = x.shape
    N = H * W
    xr = x.reshape(B, IC, N)

    n0 = len(_LAYERS[0])
    Wi = jnp.transpose(W_in, (0, 2, 1)).reshape(n0 * _C, IC)
    bi = b_in.reshape(n0 * _C, 1)

    Wls, bls = [], []
    for li, pls in enumerate(_PLACEMENTS):
        s_prev = len(_LAYERS[li]) * _C
        s_cur = len(_LAYERS[li + 1]) * _C
        Wt = jnp.zeros((s_cur, s_prev), dtype=W_hid.dtype)
        for (j, pi, h) in pls:
            Wt = jax.lax.dynamic_update_slice(Wt, W_hid[h].T, (j * _C, pi * _C))
        h0, h1 = _HID_RANGES[li]
        Wls.append(Wt)
        bls.append(b_hid[h0:h1].reshape(s_cur, 1))

    n_out = len(_LAYERS[-1])
    Wfb = jnp.concatenate([W_fc] * n_out, axis=0) * (1.0 / n_out)
    bfc = b_fc.reshape(1, -1)

    nc = W_fc.shape[1]
    full = lambda arr: pl.BlockSpec(arr.shape, lambda b: (0,) * arr.ndim)
    in_specs = [
        pl.BlockSpec((1, IC, N), lambda b: (b, 0, 0)),
        full(Wi), full(bi),
        full(Wls[0]), full(bls[0]),
        full(Wls[1]), full(bls[1]),
        full(Wls[2]), full(bls[2]),
        full(Wfb), full(bfc),
    ]
    out = pl.pallas_call(
        _body,
        grid=(B,),
        in_specs=in_specs,
        out_specs=pl.BlockSpec((1, 1, nc), lambda b: (b, 0, 0)),
        out_shape=jax.ShapeDtypeStruct((B, 1, nc), jnp.float32),
    )(xr, Wi, bi, Wls[0], bls[0], Wls[1], bls[1], Wls[2], bls[2], Wfb, bfc)
    return out.reshape(B, nc)


# speedup vs baseline: 4.3236x; 1.1223x over previous
import numpy as np
import jax
import jax.numpy as jnp
from jax.experimental import pallas as pl

_C = 32
_LAYER_SIZES = [4, 8, 8, 4]


def _dag():
    rng = np.random.RandomState(0)
    layers = []
    nid = 0
    for s in _LAYER_SIZES:
        layers.append(list(range(nid, nid + s)))
        nid += s
    parents = {}
    for l in range(1, len(layers)):
        for n in layers[l]:
            parents[n] = sorted(
                rng.choice(layers[l - 1], size=2, replace=False).tolist()
            )
    return layers, parents


_LAYERS, _PARENTS = _dag()

_PLACEMENTS = []
_HID_RANGES = []
_hid = 0
for _l in range(1, len(_LAYERS)):
    _start_prev = _LAYERS[_l - 1][0]
    _pls = []
    _h0 = _hid
    for _j, _n in enumerate(_LAYERS[_l]):
        for _p in _PARENTS[_n]:
            _pls.append((_j, _p - _start_prev, _hid))
        _hid += 1
    _PLACEMENTS.append(_pls)
    _HID_RANGES.append((_h0, _hid))


def _body(x_ref, wi_ref, bi_ref, w1_ref, b1_ref, w2_ref, b2_ref,
          w3_ref, b3_ref, wfb_ref, bfc_ref, out_ref):
    n = x_ref.shape[-1]
    bf16 = jnp.bfloat16
    xb = x_ref[0]
    a = jnp.dot(wi_ref[...], xb, preferred_element_type=jnp.float32)
    a = jnp.maximum(a + bi_ref[...], 0.0).astype(bf16)
    a = jnp.dot(w1_ref[...], a, preferred_element_type=jnp.float32)
    a = jnp.maximum(a + b1_ref[...], 0.0).astype(bf16)
    a = jnp.dot(w2_ref[...], a, preferred_element_type=jnp.float32)
    a = jnp.maximum(a + b2_ref[...], 0.0).astype(bf16)
    a = jnp.dot(w3_ref[...], a, preferred_element_type=jnp.float32)
    a = jnp.maximum(a + b3_ref[...], 0.0)
    pooled = jnp.sum(a, axis=1, keepdims=True) * (1.0 / n)
    logits = jax.lax.dot_general(
        pooled, wfb_ref[...], (((0,), (0,)), ((), ())),
        preferred_element_type=jnp.float32)
    out_ref[0] = logits + bfc_ref[...]


def kernel(x, W_in, b_in, W_hid, b_hid, W_fc, b_fc):
    B, IC, H, W = x.shape
    N = H * W
    xr = x.reshape(B, IC, N).astype(jnp.bfloat16)

    n0 = len(_LAYERS[0])
    Wi = jnp.transpose(W_in, (0, 2, 1)).reshape(n0 * _C, IC).astype(jnp.bfloat16)
    bi = b_in.reshape(n0 * _C, 1)

    Wls, bls = [], []
    for li, pls in enumerate(_PLACEMENTS):
        s_prev = len(_LAYERS[li]) * _C
        s_cur = len(_LAYERS[li + 1]) * _C
        Wt = jnp.zeros((s_cur, s_prev), dtype=W_hid.dtype)
        for (j, pi, h) in pls:
            Wt = jax.lax.dynamic_update_slice(Wt, W_hid[h].T, (j * _C, pi * _C))
        h0, h1 = _HID_RANGES[li]
        Wls.append(Wt.astype(jnp.bfloat16))
        bls.append(b_hid[h0:h1].reshape(s_cur, 1))

    n_out = len(_LAYERS[-1])
    Wfb = jnp.concatenate([W_fc] * n_out, axis=0) * (1.0 / n_out)
    bfc = b_fc.reshape(1, -1)

    nc = W_fc.shape[1]
    full = lambda arr: pl.BlockSpec(arr.shape, lambda b: (0,) * arr.ndim)
    in_specs = [
        pl.BlockSpec((1, IC, N), lambda b: (b, 0, 0)),
        full(Wi), full(bi),
        full(Wls[0]), full(bls[0]),
        full(Wls[1]), full(bls[1]),
        full(Wls[2]), full(bls[2]),
        full(Wfb), full(bfc),
    ]
    out = pl.pallas_call(
        _body,
        grid=(B,),
        in_specs=in_specs,
        out_specs=pl.BlockSpec((1, 1, nc), lambda b: (b, 0, 0)),
        out_shape=jax.ShapeDtypeStruct((B, 1, nc), jnp.float32),
    )(xr, Wi, bi, Wls[0], bls[0], Wls[1], bls[1], Wls[2], bls[2], Wfb, bfc)
    return out.reshape(B, nc)
